# baseline (speedup 1.0000x reference)
import jax
import jax.numpy as jnp
from jax.experimental import pallas as pl
from jax.experimental.pallas import tpu as pltpu

_THRESHOLD = 3e-05
_COEF = 0.01
_CHANNELS = 64.0
_ROWS = 8192
_BLOCK = 1024
_GRID = _ROWS // _BLOCK


def _sparse_max_loss_kernel(x_ref, xh_ref, o_ref, acc_ref):
    i = pl.program_id(0)
    x = x_ref[...]
    xh = xh_ref[...]

    s_r = jnp.sum(x, axis=1, keepdims=True)
    q_r = jnp.sum(x * x, axis=1, keepdims=True)
    s_c = jnp.sum(xh, axis=1)
    q_c = jnp.sum(xh * xh, axis=1)

    g = jax.lax.dot_general(
        x, xh, (((1,), (1,)), ((), ())),
        preferred_element_type=jnp.float32,
    )

    inv = 1.0 / (_CHANNELS * _CHANNELS)
    row_term = _CHANNELS - s_r * (2.0 / _CHANNELS) + q_r * inv
    col_term = (q_c * inv - s_c * (2.0 / _CHANNELS))[None, :]
    contrib = (row_term + col_term) + g * (2.0 * inv)
    masked = jnp.where(x > _THRESHOLD, contrib, 0.0)
    partial = jnp.sum(masked, keepdims=True)

    @pl.when(i == 0)
    def _init():
        acc_ref[...] = partial

    @pl.when(i > 0)
    def _accum():
        acc_ref[...] += partial

    @pl.when(i == _GRID - 1)
    def _finish():
        o_ref[...] = (_COEF / _CHANNELS) * jnp.sqrt(acc_ref[...])


def kernel(x):
    out = pl.pallas_call(
        _sparse_max_loss_kernel,
        grid=(_GRID,),
        in_specs=[
            pl.BlockSpec((_BLOCK, 64), lambda i: (i, 0)),
            pl.BlockSpec((64, 64), lambda i: (0, 0)),
        ],
        out_shape=jax.ShapeDtypeStruct((1, 1), jnp.float32),
        out_specs=pl.BlockSpec((1, 1), lambda i: (0, 0)),
        scratch_shapes=[pltpu.VMEM((1, 1), jnp.float32)],
    )(x, x)
    return jnp.reshape(out, ())


# speedup vs baseline: 1.3792x; 1.3792x over previous
import jax
import jax.numpy as jnp
from jax.experimental import pallas as pl

_THRESHOLD = 3e-05
_COEF = 0.01
_CHANNELS = 64.0


def _sparse_max_loss_kernel(x_ref, o_ref):
    x = x_ref[...]
    xh = x[:64, :]

    condf = (x > _THRESHOLD).astype(jnp.float32)
    xsq = x * x

    tn = (((0,), (0,)), ((), ()))
    a = jax.lax.dot_general(condf, x, tn, preferred_element_type=jnp.float32)
    b = jax.lax.dot_general(condf, xsq, tn, preferred_element_type=jnp.float32)

    m = jnp.sum(condf, axis=0, keepdims=True)
    xht = xh.T
    s_h = jnp.sum(xht, axis=0, keepdims=True)
    q_h = jnp.sum(xht * xht, axis=0, keepdims=True)

    total = (
        _CHANNELS * jnp.sum(m, keepdims=True)
        - (jnp.sum(a, keepdims=True) + jnp.sum(m * s_h, keepdims=True))
        * (1.0 / 32.0)
        + (jnp.sum(b, keepdims=True) + jnp.sum(m * q_h, keepdims=True))
        * (1.0 / 4096.0)
        + jnp.sum(a * xh, keepdims=True) * (1.0 / 2048.0)
    )
    o_ref[...] = (_COEF / _CHANNELS) * jnp.sqrt(total)


def kernel(x):
    out = pl.pallas_call(
        _sparse_max_loss_kernel,
        out_shape=jax.ShapeDtypeStruct((1, 1), jnp.float32),
    )(x)
    return jnp.reshape(out, ())


# speedup vs baseline: 1.4376x; 1.0424x over previous
import jax
import jax.numpy as jnp
from jax.experimental import pallas as pl
from jax.experimental.pallas import tpu as pltpu

_THRESHOLD = 3e-05
_COEF = 0.01
_CHANNELS = 64.0
_ROWS = 8192
_NCHUNK = 4
_CHUNK = _ROWS // _NCHUNK


def _sparse_max_loss_kernel(x_hbm, o_ref, buf, sem):
    copies = []
    for i in range(_NCHUNK):
        rows = pl.ds(i * _CHUNK, _CHUNK)
        c = pltpu.make_async_copy(x_hbm.at[rows, :], buf.at[rows, :], sem.at[i])
        c.start()
        copies.append(c)

    tn = (((0,), (0,)), ((), ()))
    a = jnp.zeros((64, 64), jnp.float32)
    b = jnp.zeros((64, 64), jnp.float32)
    m = jnp.zeros((1, 64), jnp.float32)
    xh = None
    for i in range(_NCHUNK):
        copies[i].wait()
        x = buf[pl.ds(i * _CHUNK, _CHUNK), :]
        if i == 0:
            xh = x[:64, :]
        condf = (x > _THRESHOLD).astype(jnp.float32)
        xsq = x * x
        a += jax.lax.dot_general(condf, x, tn, preferred_element_type=jnp.float32)
        b += jax.lax.dot_general(condf, xsq, tn, preferred_element_type=jnp.float32)
        m += jnp.sum(condf, axis=0, keepdims=True)

    xht = xh.T
    s_h = jnp.sum(xht, axis=0, keepdims=True)
    q_h = jnp.sum(xht * xht, axis=0, keepdims=True)

    total = (
        _CHANNELS * jnp.sum(m, keepdims=True)
        - (jnp.sum(a, keepdims=True) + jnp.sum(m * s_h, keepdims=True))
        * (1.0 / 32.0)
        + (jnp.sum(b, keepdims=True) + jnp.sum(m * q_h, keepdims=True))
        * (1.0 / 4096.0)
        + jnp.sum(a * xh, keepdims=True) * (1.0 / 2048.0)
    )
    o_ref[...] = (_COEF / _CHANNELS) * jnp.sqrt(total)


def kernel(x):
    out = pl.pallas_call(
        _sparse_max_loss_kernel,
        in_specs=[pl.BlockSpec(memory_space=pltpu.MemorySpace.HBM)],
        out_shape=jax.ShapeDtypeStruct((1, 1), jnp.float32),
        scratch_shapes=[
            pltpu.VMEM((_ROWS, 64), jnp.float32),
            pltpu.SemaphoreType.DMA((_NCHUNK,)),
        ],
    )(x)
    return jnp.reshape(out, ())
